# baseline (speedup 1.0000x reference)
import functools

import jax
import jax.numpy as jnp
from jax import lax
from jax.experimental import pallas as pl
from jax.experimental.pallas import tpu as pltpu
from jax.experimental.pallas import tpu_sc as plsc

B, N, K = 2, 16384, 16
C = 64
ROW = 128
XP = 4
XOFF = C + C // 2
NK = N * K
BN = B * N
BNK = B * NK
NW = 32
CH = 128
PERW = BNK // NW
NCH = PERW // CH
NBUF = 2
BLK1 = 1024
BLK3 = 512
M3 = BLK3 * K


def _pack_bf16_pairs(v):
    lo = lax.convert_element_type(
        lax.bitcast_convert_type(v[:, :C // 2].astype(jnp.bfloat16), jnp.uint16),
        jnp.uint32)
    hi = lax.convert_element_type(
        lax.bitcast_convert_type(v[:, C // 2:].astype(jnp.bfloat16), jnp.uint16),
        jnp.uint32)
    return lax.bitcast_convert_type(lo | (hi << 16), jnp.float32)


def _unpack_bf16_pairs(packed_f32):
    w = lax.bitcast_convert_type(packed_f32, jnp.uint32)
    lo = lax.convert_element_type(w & jnp.uint32(0xFFFF), jnp.uint16)
    hi = lax.convert_element_type(w >> 16, jnp.uint16)
    return jnp.concatenate(
        [lax.bitcast_convert_type(lo, jnp.bfloat16).astype(jnp.float32),
         lax.bitcast_convert_type(hi, jnp.bfloat16).astype(jnp.float32)],
        axis=1)


def _proj_body(f_ref, x_ref, wkt_ref, bk_ref, wvt_ref, bv_ref, tab_ref):
    f = f_ref[...]
    k = jnp.dot(f, wkt_ref[...], preferred_element_type=jnp.float32) + bk_ref[...]
    v = jnp.dot(f, wvt_ref[...], preferred_element_type=jnp.float32) + bv_ref[...]
    vpack = _pack_bf16_pairs(v)
    pad = jnp.zeros((f.shape[0], ROW - XOFF - 3), jnp.float32)
    tab_ref[...] = jnp.concatenate([k, vpack, x_ref[...], pad], axis=1)


def _make_proj():
    grid = (BN // BLK1,)
    full = lambda shape: pl.BlockSpec(shape, lambda i: (0, 0))
    return pl.pallas_call(
        _proj_body,
        grid=grid,
        in_specs=[
            pl.BlockSpec((BLK1, C), lambda i: (i, 0)),
            pl.BlockSpec((BLK1, 3), lambda i: (i, 0)),
            full((C, C)), full((1, C)), full((C, C)), full((1, C)),
        ],
        out_specs=pl.BlockSpec((BLK1, ROW), lambda i: (i, 0)),
        out_shape=jax.ShapeDtypeStruct((BN, ROW), jnp.float32),
    )


def _gather_body(tab_hbm, idx_hbm, out_hbm, idx_v, *bufs_and_sems):
    bufs = bufs_and_sems[:NBUF]
    sgs = bufs_and_sems[NBUF:2 * NBUF]
    sws = bufs_and_sems[2 * NBUF:3 * NBUF]
    wid = lax.axis_index("s") * 2 + lax.axis_index("c")
    base = wid * PERW
    pltpu.sync_copy(idx_hbm.at[pl.ds(base, PERW)], idx_v)
    off = jnp.where(wid >= NW // 2, jnp.int32(N), jnp.int32(0))

    def add_off(i, carry):
        idx_v[pl.ds(i * 16, 16)] = idx_v[pl.ds(i * 16, 16)] + off
        return carry

    lax.fori_loop(0, PERW // 16, add_off, 0)

    def gath_start(c, b):
        pltpu.async_copy(tab_hbm.at[idx_v.at[pl.ds(c * CH, CH)]], bufs[b], sgs[b])

    def gath_wait(c, b):
        pltpu.make_async_copy(tab_hbm.at[idx_v.at[pl.ds(c * CH, CH)]],
                              bufs[b], sgs[b]).wait()

    def wb_start(c, b):
        pltpu.async_copy(bufs[b], out_hbm.at[pl.ds(base + c * CH, CH)], sws[b])

    def wb_wait(c, b):
        pltpu.make_async_copy(bufs[b], out_hbm.at[pl.ds(base + c * CH, CH)],
                              sws[b]).wait()

    for b in range(NBUF):
        gath_start(b, b)

    def body(j, carry):
        c0 = NBUF * j
        for b in range(NBUF):
            gath_wait(c0 + b, b)
            wb_start(c0 + b, b)

        @pl.when(c0 + NBUF < NCH)
        def _():
            for b in range(NBUF):
                wb_wait(c0 + b, b)
                gath_start(c0 + NBUF + b, b)

        return carry

    lax.fori_loop(0, NCH // NBUF, body, 0)
    for b in range(NBUF):
        wb_wait(NCH - NBUF + b, b)


def _make_gather():
    mesh = plsc.VectorSubcoreMesh(core_axis_name="c", subcore_axis_name="s")
    return pl.kernel(
        _gather_body,
        out_type=jax.ShapeDtypeStruct((BNK, ROW), jnp.float32),
        name="neighbor_row_gather",
        mesh=mesh,
        scratch_types=[pltpu.VMEM((PERW,), jnp.int32)]
        + [pltpu.VMEM((CH, ROW), jnp.float32) for _ in range(NBUF)]
        + [pltpu.SemaphoreType.DMA for _ in range(2 * NBUF)],
    )


def _fused_body(f_ref, x_ref, g_ref,
                wqt_ref, bq2_ref, wd1p_ref, wd2t_ref,
                wg1t_ref, bg1_ref, wg2t_ref, wot_ref, bo2_ref, o_ref):
    dot = functools.partial(jnp.dot, preferred_element_type=jnp.float32)
    f = f_ref[...]
    q = dot(f, wqt_ref[...]) + bq2_ref[...]
    xi = x_ref[...]
    rows = g_ref[...]
    kg = rows[:, :C]
    vg = _unpack_bf16_pairs(rows[:, C:XOFF])
    xg = rows[:, XOFF:XOFF + XP]
    wd1p = wd1p_ref[...]
    pn = dot(xg, wd1p)
    pq = dot(xi, wd1p)
    pre = pq.reshape(BLK3, 1, C) - pn.reshape(BLK3, K, C)
    pe = dot(jnp.maximum(pre, 0.0).reshape(M3, C), wd2t_ref[...])
    af = q.reshape(BLK3, 1, C) - kg.reshape(BLK3, K, C) + pe.reshape(BLK3, K, C)
    h1 = jnp.maximum(dot(af.reshape(M3, C), wg1t_ref[...]) + bg1_ref[...], 0.0)
    e = jnp.exp(dot(h1, wg2t_ref[...])).reshape(BLK3, K, C)
    den = jnp.sum(e, axis=1)
    num = jnp.sum(e * (vg.reshape(BLK3, K, C) + pe.reshape(BLK3, K, C)), axis=1)
    o_ref[...] = dot(num / den, wot_ref[...]) + bo2_ref[...]


def _make_fused():
    grid = (BN // BLK3,)
    full = lambda shape: pl.BlockSpec(shape, lambda i: (0, 0))
    return pl.pallas_call(
        _fused_body,
        grid=grid,
        in_specs=[
            pl.BlockSpec((BLK3, C), lambda i: (i, 0)),
            pl.BlockSpec((BLK3, XP), lambda i: (i, 0)),
            pl.BlockSpec((M3, ROW), lambda i: (i, 0)),
            full((C, C)), full((1, C)),
            full((XP, C)),
            full((C, C)),
            full((C, C)), full((1, C)),
            full((C, C)),
            full((C, C)), full((1, C)),
        ],
        out_specs=pl.BlockSpec((BLK3, C), lambda i: (i, 0)),
        out_shape=jax.ShapeDtypeStruct((BN, C), jnp.float32),
    )


def kernel(xyz, features, neighbor_indices, Wq, bq, Wk, bk, Wv, bv,
           Wd1, bd1, Wd2, bd2, Wg1, bg1, Wg2, bg2, Wo, bo):
    f32 = jnp.float32
    idx = neighbor_indices.astype(jnp.int32).reshape(BNK)
    xyz = xyz.astype(f32).reshape(BN, 3)
    xyzp = jnp.concatenate([xyz, jnp.ones((BN, XP - 3), f32)], axis=-1)
    wqt, wkt, wvt = Wq.T, Wk.T, Wv.T
    wd1p = jnp.zeros((XP, C), f32).at[:3, :].set(Wd1.T).at[3, :].set(bd1)
    wd2t, wg1t, wg2t, wot = Wd2.T, Wg1.T, Wg2.T, Wo.T
    bq2 = bq + bd2
    bo2 = bd2 @ wot + bo
    row = lambda b: b.reshape(1, C).astype(f32)

    fv = features.reshape(BN, C)
    tab = _make_proj()(fv, xyz, wkt, row(bk), wvt, row(bv))
    g = _make_gather()(tab, idx)
    out = _make_fused()(fv, xyzp, g,
                        wqt, row(bq2), wd1p, wd2t,
                        wg1t, row(bg1), wg2t, wot, row(bo2))
    return out.reshape(B, N, C)


# speedup vs baseline: 1.0816x; 1.0816x over previous
import functools

import jax
import jax.numpy as jnp
from jax import lax
from jax.experimental import pallas as pl
from jax.experimental.pallas import tpu as pltpu
from jax.experimental.pallas import tpu_sc as plsc

B, N, K = 2, 16384, 16
C = 64
ROW = 128
XP = 4
XOFF = C + C // 2
NK = N * K
NW = 32
CH = 128
HALF = NK // 2
PERW = HALF // NW
NCH = PERW // CH
BLK1 = 1024
BLK3 = 512
M3 = BLK3 * K


def _pack_bf16_pairs(v):
    lo = lax.convert_element_type(
        lax.bitcast_convert_type(v[:, :C // 2].astype(jnp.bfloat16), jnp.uint16),
        jnp.uint32)
    hi = lax.convert_element_type(
        lax.bitcast_convert_type(v[:, C // 2:].astype(jnp.bfloat16), jnp.uint16),
        jnp.uint32)
    return lax.bitcast_convert_type(lo | (hi << 16), jnp.float32)


def _unpack_bf16_pairs(packed_f32):
    w = lax.bitcast_convert_type(packed_f32, jnp.uint32)
    lo = lax.convert_element_type(w & jnp.uint32(0xFFFF), jnp.uint16)
    hi = lax.convert_element_type(w >> 16, jnp.uint16)
    return jnp.concatenate(
        [lax.bitcast_convert_type(lo, jnp.bfloat16).astype(jnp.float32),
         lax.bitcast_convert_type(hi, jnp.bfloat16).astype(jnp.float32)],
        axis=1)


def _proj_body(f_ref, x_ref, wkt_ref, bk_ref, wvt_ref, bv_ref, tab_ref):
    f = f_ref[...]
    k = jnp.dot(f, wkt_ref[...], preferred_element_type=jnp.float32) + bk_ref[...]
    v = jnp.dot(f, wvt_ref[...], preferred_element_type=jnp.float32) + bv_ref[...]
    vpack = _pack_bf16_pairs(v)
    pad = jnp.zeros((f.shape[0], ROW - XOFF - 3), jnp.float32)
    tab_ref[...] = jnp.concatenate([k, vpack, x_ref[...], pad], axis=1)


def _make_proj():
    grid = (N // BLK1,)
    full = lambda shape: pl.BlockSpec(shape, lambda i: (0, 0))
    return pl.pallas_call(
        _proj_body,
        grid=grid,
        in_specs=[
            pl.BlockSpec((BLK1, C), lambda i: (i, 0)),
            pl.BlockSpec((BLK1, 3), lambda i: (i, 0)),
            full((C, C)), full((1, C)), full((C, C)), full((1, C)),
        ],
        out_specs=pl.BlockSpec((BLK1, ROW), lambda i: (i, 0)),
        out_shape=jax.ShapeDtypeStruct((N, ROW), jnp.float32),
    )


NBUF = 2


def _gather_body(tab_hbm, idx_hbm, out_hbm, idx_v, *bufs_and_sems):
    bufs = bufs_and_sems[:NBUF]
    sgs = bufs_and_sems[NBUF:2 * NBUF]
    sws = bufs_and_sems[2 * NBUF:3 * NBUF]
    wid = lax.axis_index("s") * 2 + lax.axis_index("c")
    pltpu.sync_copy(idx_hbm.at[pl.ds(wid * NCH, NCH)], idx_v)
    base = wid * PERW

    def gath_start(c, b):
        pltpu.async_copy(tab_hbm.at[idx_v.at[c]], bufs[b], sgs[b])

    def gath_wait(c, b):
        pltpu.make_async_copy(tab_hbm.at[idx_v.at[c]], bufs[b], sgs[b]).wait()

    def wb_start(c, b):
        pltpu.async_copy(bufs[b], out_hbm.at[pl.ds(base + c * CH, CH)], sws[b])

    def wb_wait(c, b):
        pltpu.make_async_copy(bufs[b], out_hbm.at[pl.ds(base + c * CH, CH)],
                              sws[b]).wait()

    for b in range(NBUF):
        gath_start(b, b)

    def body(j, carry):
        c0 = NBUF * j
        for b in range(NBUF):
            gath_wait(c0 + b, b)
            wb_start(c0 + b, b)

        @pl.when(c0 + NBUF < NCH)
        def _():
            for b in range(NBUF):
                wb_wait(c0 + b, b)
                gath_start(c0 + NBUF + b, b)

        return carry

    lax.fori_loop(0, NCH // NBUF, body, 0)
    for b in range(NBUF):
        wb_wait(NCH - NBUF + b, b)


def _make_gather():
    mesh = plsc.VectorSubcoreMesh(core_axis_name="c", subcore_axis_name="s")
    return pl.kernel(
        _gather_body,
        out_type=jax.ShapeDtypeStruct((HALF, ROW), jnp.float32),
        mesh=mesh,
        scratch_types=[pltpu.VMEM((NCH, CH), jnp.int32)]
        + [pltpu.VMEM((CH, ROW), jnp.float32) for _ in range(NBUF)]
        + [pltpu.SemaphoreType.DMA for _ in range(2 * NBUF)],
    )


def _fused_body(f_ref, x_ref, g_ref,
                wqt_ref, bq2_ref, wd1p_ref, wd2t_ref,
                wg1t_ref, bg1_ref, wg2t_ref, wot_ref, bo2_ref, o_ref):
    dot = functools.partial(jnp.dot, preferred_element_type=jnp.float32)
    f = f_ref[...]
    q = dot(f, wqt_ref[...]) + bq2_ref[...]
    xi = x_ref[...]
    rows = g_ref[...]
    kg = rows[:, :C]
    vg = _unpack_bf16_pairs(rows[:, C:XOFF])
    xg = rows[:, XOFF:XOFF + XP]
    wd1p = wd1p_ref[...]
    pn = dot(xg, wd1p)
    pq = dot(xi, wd1p)
    pre = pq.reshape(BLK3, 1, C) - pn.reshape(BLK3, K, C)
    pe = dot(jnp.maximum(pre, 0.0).reshape(M3, C), wd2t_ref[...])
    af = q.reshape(BLK3, 1, C) - kg.reshape(BLK3, K, C) + pe.reshape(BLK3, K, C)
    h1 = jnp.maximum(dot(af.reshape(M3, C), wg1t_ref[...]) + bg1_ref[...], 0.0)
    e = jnp.exp(dot(h1, wg2t_ref[...])).reshape(BLK3, K, C)
    den = jnp.sum(e, axis=1)
    num = jnp.sum(e * (vg.reshape(BLK3, K, C) + pe.reshape(BLK3, K, C)), axis=1)
    o_ref[...] = dot(num / den, wot_ref[...]) + bo2_ref[...]


def _make_fused():
    grid = (N // 2 // BLK3,)
    full = lambda shape: pl.BlockSpec(shape, lambda i: (0, 0))
    return pl.pallas_call(
        _fused_body,
        grid=grid,
        in_specs=[
            pl.BlockSpec((BLK3, C), lambda i: (i, 0)),
            pl.BlockSpec((BLK3, XP), lambda i: (i, 0)),
            pl.BlockSpec((M3, ROW), lambda i: (i, 0)),
            full((C, C)), full((1, C)),
            full((XP, C)),
            full((C, C)),
            full((C, C)), full((1, C)),
            full((C, C)),
            full((C, C)), full((1, C)),
        ],
        out_specs=pl.BlockSpec((BLK3, C), lambda i: (i, 0)),
        out_shape=jax.ShapeDtypeStruct((N // 2, C), jnp.float32),
    )


def kernel(xyz, features, neighbor_indices, Wq, bq, Wk, bk, Wv, bv,
           Wd1, bd1, Wd2, bd2, Wg1, bg1, Wg2, bg2, Wo, bo):
    f32 = jnp.float32
    idx = neighbor_indices.astype(jnp.int32)
    xyz = xyz.astype(f32)
    xyzp = jnp.concatenate([xyz, jnp.ones((B, N, XP - 3), f32)], axis=-1)
    wqt, wkt, wvt = Wq.T, Wk.T, Wv.T
    wd1p = jnp.zeros((XP, C), f32).at[:3, :].set(Wd1.T).at[3, :].set(bd1)
    wd2t, wg1t, wg2t, wot = Wd2.T, Wg1.T, Wg2.T, Wo.T
    bq2 = bq + bd2
    bo2 = bd2 @ wot + bo
    row = lambda b: b.reshape(1, C).astype(f32)

    proj = _make_proj()
    gather = _make_gather()
    fused = _make_fused()

    Nh = N // 2
    tabs = [proj(features[b], xyz[b], wkt, row(bk), wvt, row(bv))
            for b in range(B)]
    idxr = idx.reshape(B, NK // CH, CH)
    gs = [gather(tabs[b], idxr[b, h * (HALF // CH):(h + 1) * (HALF // CH)])
          for b in range(B) for h in range(2)]
    outs = [fused(features[b, h * Nh:(h + 1) * Nh],
                  xyzp[b, h * Nh:(h + 1) * Nh], gs[2 * b + h],
                  wqt, row(bq2), wd1p, wd2t,
                  wg1t, row(bg1), wg2t, wot, row(bo2))
            for b in range(B) for h in range(2)]
    return jnp.stack([jnp.concatenate([outs[2 * b], outs[2 * b + 1]])
                      for b in range(B)])


# speedup vs baseline: 1.1286x; 1.0434x over previous
import functools

import jax
import jax.numpy as jnp
from jax import lax
from jax.experimental import pallas as pl
from jax.experimental.pallas import tpu as pltpu
from jax.experimental.pallas import tpu_sc as plsc

B, N, K = 2, 16384, 16
C = 64
ROW = 128
XP = 4
XOFF = C + C // 2
NK = N * K
NW = 32
CH = 128
PERW = NK // NW
NCH = PERW // CH
BLK1 = 1024
BLK3 = 512
M3 = BLK3 * K


def _pack_bf16_pairs(v):
    lo = lax.convert_element_type(
        lax.bitcast_convert_type(v[:, :C // 2].astype(jnp.bfloat16), jnp.uint16),
        jnp.uint32)
    hi = lax.convert_element_type(
        lax.bitcast_convert_type(v[:, C // 2:].astype(jnp.bfloat16), jnp.uint16),
        jnp.uint32)
    return lax.bitcast_convert_type(lo | (hi << 16), jnp.float32)


def _unpack_bf16_pairs(packed_f32):
    w = lax.bitcast_convert_type(packed_f32, jnp.uint32)
    lo = lax.convert_element_type(w & jnp.uint32(0xFFFF), jnp.uint16)
    hi = lax.convert_element_type(w >> 16, jnp.uint16)
    return jnp.concatenate(
        [lax.bitcast_convert_type(lo, jnp.bfloat16).astype(jnp.float32),
         lax.bitcast_convert_type(hi, jnp.bfloat16).astype(jnp.float32)],
        axis=1)


def _proj_body(f_ref, x_ref, wkt_ref, bk_ref, wvt_ref, bv_ref, tab_ref):
    f = f_ref[...]
    k = jnp.dot(f, wkt_ref[...], preferred_element_type=jnp.float32) + bk_ref[...]
    v = jnp.dot(f, wvt_ref[...], preferred_element_type=jnp.float32) + bv_ref[...]
    vpack = _pack_bf16_pairs(v)
    pad = jnp.zeros((f.shape[0], ROW - XOFF - 3), jnp.float32)
    tab_ref[...] = jnp.concatenate([k, vpack, x_ref[...], pad], axis=1)


def _make_proj():
    grid = (N // BLK1,)
    full = lambda shape: pl.BlockSpec(shape, lambda i: (0, 0))
    return pl.pallas_call(
        _proj_body,
        grid=grid,
        in_specs=[
            pl.BlockSpec((BLK1, C), lambda i: (i, 0)),
            pl.BlockSpec((BLK1, 3), lambda i: (i, 0)),
            full((C, C)), full((1, C)), full((C, C)), full((1, C)),
        ],
        out_specs=pl.BlockSpec((BLK1, ROW), lambda i: (i, 0)),
        out_shape=jax.ShapeDtypeStruct((N, ROW), jnp.float32),
    )


NBUF = 2


def _gather_body(tab_hbm, idx_hbm, out_hbm, idx_v, *bufs_and_sems):
    bufs = bufs_and_sems[:NBUF]
    sgs = bufs_and_sems[NBUF:2 * NBUF]
    sws = bufs_and_sems[2 * NBUF:3 * NBUF]
    wid = lax.axis_index("s") * 2 + lax.axis_index("c")
    pltpu.sync_copy(idx_hbm.at[pl.ds(wid * NCH, NCH)], idx_v)
    base = wid * PERW

    def gath_start(c, b):
        pltpu.async_copy(tab_hbm.at[idx_v.at[c]], bufs[b], sgs[b])

    def gath_wait(c, b):
        pltpu.make_async_copy(tab_hbm.at[idx_v.at[c]], bufs[b], sgs[b]).wait()

    def wb_start(c, b):
        pltpu.async_copy(bufs[b], out_hbm.at[pl.ds(base + c * CH, CH)], sws[b])

    def wb_wait(c, b):
        pltpu.make_async_copy(bufs[b], out_hbm.at[pl.ds(base + c * CH, CH)],
                              sws[b]).wait()

    for b in range(NBUF):
        gath_start(b, b)

    def body(j, carry):
        c0 = NBUF * j
        for b in range(NBUF):
            gath_wait(c0 + b, b)
            wb_start(c0 + b, b)

        @pl.when(c0 + NBUF < NCH)
        def _():
            for b in range(NBUF):
                wb_wait(c0 + b, b)
                gath_start(c0 + NBUF + b, b)

        return carry

    lax.fori_loop(0, NCH // NBUF, body, 0)
    for b in range(NBUF):
        wb_wait(NCH - NBUF + b, b)


def _make_gather():
    mesh = plsc.VectorSubcoreMesh(core_axis_name="c", subcore_axis_name="s")
    return pl.kernel(
        _gather_body,
        out_type=jax.ShapeDtypeStruct((NK, ROW), jnp.float32),
        mesh=mesh,
        scratch_types=[pltpu.VMEM((NCH, CH), jnp.int32)]
        + [pltpu.VMEM((CH, ROW), jnp.float32) for _ in range(NBUF)]
        + [pltpu.SemaphoreType.DMA for _ in range(2 * NBUF)],
    )


def _fused_body(f_ref, x_ref, g_ref,
                wqt_ref, bq2_ref, wd1p_ref, wd2t_ref,
                wg1t_ref, bg1_ref, wg2t_ref, wot_ref, bo2_ref, o_ref):
    dot = functools.partial(jnp.dot, preferred_element_type=jnp.float32)
    f = f_ref[...]
    q = dot(f, wqt_ref[...]) + bq2_ref[...]
    xi = x_ref[...]
    rows = g_ref[...]
    kg = rows[:, :C]
    vg = _unpack_bf16_pairs(rows[:, C:XOFF])
    xg = rows[:, XOFF:XOFF + XP]
    wd1p = wd1p_ref[...]
    pn = dot(xg, wd1p)
    pq = dot(xi, wd1p)
    pre = pq.reshape(BLK3, 1, C) - pn.reshape(BLK3, K, C)
    pe = dot(jnp.maximum(pre, 0.0).reshape(M3, C), wd2t_ref[...])
    af = q.reshape(BLK3, 1, C) - kg.reshape(BLK3, K, C) + pe.reshape(BLK3, K, C)
    h1 = jnp.maximum(dot(af.reshape(M3, C), wg1t_ref[...]) + bg1_ref[...], 0.0)
    e = jnp.exp(dot(h1, wg2t_ref[...])).reshape(BLK3, K, C)
    den = jnp.sum(e, axis=1)
    num = jnp.sum(e * (vg.reshape(BLK3, K, C) + pe.reshape(BLK3, K, C)), axis=1)
    o_ref[...] = dot(num / den, wot_ref[...]) + bo2_ref[...]


def _make_fused():
    grid = (N // BLK3,)
    full = lambda shape: pl.BlockSpec(shape, lambda i: (0, 0))
    return pl.pallas_call(
        _fused_body,
        grid=grid,
        in_specs=[
            pl.BlockSpec((BLK3, C), lambda i: (i, 0)),
            pl.BlockSpec((BLK3, XP), lambda i: (i, 0)),
            pl.BlockSpec((M3, ROW), lambda i: (i, 0)),
            full((C, C)), full((1, C)),
            full((XP, C)),
            full((C, C)),
            full((C, C)), full((1, C)),
            full((C, C)),
            full((C, C)), full((1, C)),
        ],
        out_specs=pl.BlockSpec((BLK3, C), lambda i: (i, 0)),
        out_shape=jax.ShapeDtypeStruct((N, C), jnp.float32),
    )


def kernel(xyz, features, neighbor_indices, Wq, bq, Wk, bk, Wv, bv,
           Wd1, bd1, Wd2, bd2, Wg1, bg1, Wg2, bg2, Wo, bo):
    f32 = jnp.float32
    idx = neighbor_indices.astype(jnp.int32)
    xyz = xyz.astype(f32)
    xyzp = jnp.concatenate([xyz, jnp.ones((B, N, XP - 3), f32)], axis=-1)
    wqt, wkt, wvt = Wq.T, Wk.T, Wv.T
    wd1p = jnp.zeros((XP, C), f32).at[:3, :].set(Wd1.T).at[3, :].set(bd1)
    wd2t, wg1t, wg2t, wot = Wd2.T, Wg1.T, Wg2.T, Wo.T
    bq2 = bq + bd2
    bo2 = bd2 @ wot + bo
    row = lambda b: b.reshape(1, C).astype(f32)

    proj = _make_proj()
    gather = _make_gather()
    fused = _make_fused()

    tabs = [proj(features[b], xyz[b], wkt, row(bk), wvt, row(bv))
            for b in range(B)]
    gs = [gather(tabs[b], idx[b].reshape(NK // CH, CH)) for b in range(B)]
    outs = [fused(features[b], xyzp[b], gs[b],
                  wqt, row(bq2), wd1p, wd2t,
                  wg1t, row(bg1), wg2t, wot, row(bo2))
            for b in range(B)]
    return jnp.stack(outs)
